# Initial kernel scaffold; baseline (speedup 1.0000x reference)
import jax
import jax.numpy as jnp
from jax.experimental import pallas as pl


def kernel(batch_reprs, batch_labels):
    raise NotImplementedError("write your pallas kernel here")


# speedup vs baseline: 10.8417x; 10.8417x over previous
import functools

import jax
import jax.numpy as jnp
from jax import lax
from jax.experimental import pallas as pl

MARGIN = 0.2
NEG = float("-inf")


def _loss_body(x_ref, xt_ref, labc_ref, labr_ref, out_ref):
    i = pl.program_id(0)

    @pl.when(i == 0)
    def _():
        out_ref[...] = jnp.zeros_like(out_ref)

    x = x_ref[...]
    xt = xt_ref[...]
    R = x.shape[0]
    N = xt.shape[1]
    Lf = jnp.float32(N)

    g = jnp.dot(x, xt, preferred_element_type=jnp.float32)
    sqc = jnp.sum(x * x, axis=1, keepdims=True)
    sqr = jnp.sum(xt * xt, axis=0, keepdims=True)
    d2 = jnp.maximum(sqc + sqr - 2.0 * g, 0.0)
    dist = jnp.sqrt(jnp.maximum(d2, 1e-12))
    match = labc_ref[...] == labr_ref[...]
    mf = match.astype(jnp.float32)
    v = -dist + MARGIN * (1.0 - mf)

    kf = jnp.sum(mf, axis=1, keepdims=True)
    ki = kf.astype(jnp.int32)
    kmax = jnp.max(ki)
    iota = lax.broadcasted_iota(jnp.int32, (R, N), 1)
    v2_init = jnp.where(match, v, NEG)

    def step(t, carry):
        acc, v1, v2 = carry
        r = t + 1
        tf = r.astype(jnp.float32) - 1.0
        active = r <= ki
        m1 = jnp.max(v1, axis=1, keepdims=True)
        w1 = 0.5 + 0.5 * (kf - tf) / kf
        acc = acc + jnp.sum(jnp.where(active, w1 * m1, 0.0))
        i1 = jnp.min(jnp.where(v1 == m1, iota, N), axis=1, keepdims=True)
        v1 = jnp.where(iota == i1, NEG, v1)
        m2 = jnp.max(v2, axis=1, keepdims=True)
        i2 = jnp.min(jnp.where(v2 == m2, iota, N), axis=1, keepdims=True)
        cnt_gt = jnp.sum((v > m2).astype(jnp.float32), axis=1, keepdims=True)
        cnt_eq = jnp.sum(((v == m2) & (iota < i2)).astype(jnp.float32),
                         axis=1, keepdims=True)
        rank2 = cnt_gt + cnt_eq + 1.0
        wfp = 0.5 + 0.5 * (kf - rank2 + 1.0) / kf
        wfn = 0.5 + 0.5 * (rank2 - kf) / (Lf - kf)
        w2 = jnp.where(rank2 <= kf, wfp, wfn)
        acc = acc - jnp.sum(jnp.where(active, w2 * m2, 0.0))
        v2 = jnp.where(iota == i2, NEG, v2)
        return acc, v1, v2

    acc, _, _ = lax.fori_loop(0, kmax, step, (jnp.float32(0.0), v, v2_init))
    out_ref[...] = out_ref[...] + acc


@jax.jit
def kernel(batch_reprs, batch_labels):
    x = batch_reprs.astype(jnp.float32)
    n, d = x.shape
    labf = batch_labels.reshape(-1).astype(jnp.float32)
    lab_col = labf.reshape(n, 1)
    lab_row = labf.reshape(1, n)
    xt = x.T

    rows = 128 if n % 128 == 0 else n
    nblk = n // rows

    out = pl.pallas_call(
        _loss_body,
        grid=(nblk,),
        in_specs=[
            pl.BlockSpec((rows, d), lambda i: (i, 0)),
            pl.BlockSpec((d, n), lambda i: (0, 0)),
            pl.BlockSpec((rows, 1), lambda i: (i, 0)),
            pl.BlockSpec((1, n), lambda i: (0, 0)),
        ],
        out_specs=pl.BlockSpec((1, 1), lambda i: (0, 0)),
        out_shape=jax.ShapeDtypeStruct((1, 1), jnp.float32),
    )(x, xt, lab_col, lab_row)
    return out[0, 0]


# speedup vs baseline: 17.4384x; 1.6085x over previous
import functools

import jax
import jax.numpy as jnp
from jax import lax
from jax.experimental import pallas as pl

MARGIN = 0.2
NEG = float("-inf")


def _loss_body(x_ref, xt_ref, labc_ref, labr_ref, out_ref):
    i = pl.program_id(0)

    @pl.when(i == 0)
    def _():
        out_ref[...] = jnp.zeros_like(out_ref)

    x = x_ref[...]
    xt = xt_ref[...]
    R = x.shape[0]
    N = xt.shape[1]
    Lf = jnp.float32(N)

    g = jnp.dot(x, xt, preferred_element_type=jnp.float32)
    sqc = jnp.sum(x * x, axis=1, keepdims=True)
    sqr = jnp.sum(xt * xt, axis=0, keepdims=True)
    d2 = jnp.maximum(sqc + sqr - 2.0 * g, 0.0)
    dist = jnp.sqrt(jnp.maximum(d2, 1e-12))
    match = labc_ref[...] == labr_ref[...]
    mf = match.astype(jnp.float32)
    v = -dist + MARGIN * (1.0 - mf)

    kf = jnp.sum(mf, axis=1, keepdims=True)
    ki = kf.astype(jnp.int32)
    kmax = jnp.max(ki)
    iota = lax.broadcasted_iota(jnp.int32, (R, N), 1)
    v2_init = jnp.where(match, v, NEG)

    def step(t, carry):
        acc, v1, v2 = carry
        r = t + 1
        tf = r.astype(jnp.float32) - 1.0
        active = r <= ki
        m1 = jnp.max(v1, axis=1, keepdims=True)
        w1 = 0.5 + 0.5 * (kf - tf) / kf
        acc = acc + jnp.sum(jnp.where(active, w1 * m1, 0.0))
        i1 = jnp.min(jnp.where(v1 == m1, iota, N), axis=1, keepdims=True)
        v1 = jnp.where(iota == i1, NEG, v1)
        m2 = jnp.max(v2, axis=1, keepdims=True)
        i2 = jnp.min(jnp.where(v2 == m2, iota, N), axis=1, keepdims=True)
        cnt_gt = jnp.sum((v > m2).astype(jnp.float32), axis=1, keepdims=True)
        cnt_eq = jnp.sum(((v == m2) & (iota < i2)).astype(jnp.float32),
                         axis=1, keepdims=True)
        rank2 = cnt_gt + cnt_eq + 1.0
        wfp = 0.5 + 0.5 * (kf - rank2 + 1.0) / kf
        wfn = 0.5 + 0.5 * (rank2 - kf) / (Lf - kf)
        w2 = jnp.where(rank2 <= kf, wfp, wfn)
        acc = acc - jnp.sum(jnp.where(active, w2 * m2, 0.0))
        v2 = jnp.where(iota == i2, NEG, v2)
        return acc, v1, v2

    acc, _, _ = lax.fori_loop(0, kmax, step, (jnp.float32(0.0), v, v2_init))
    out_ref[...] = out_ref[...] + acc


@jax.jit
def kernel(batch_reprs, batch_labels):
    x = batch_reprs.astype(jnp.float32)
    n, d = x.shape
    labels = batch_labels.reshape(-1)
    counts = jnp.zeros((512,), jnp.int32).at[labels].add(1)
    k_row = counts[labels]
    order = jnp.argsort(k_row)
    x = x[order]
    labf = labels[order].astype(jnp.float32)
    lab_col = labf.reshape(n, 1)
    lab_row = labf.reshape(1, n)
    xt = x.T

    rows = 128 if n % 128 == 0 else n
    nblk = n // rows

    out = pl.pallas_call(
        _loss_body,
        grid=(nblk,),
        in_specs=[
            pl.BlockSpec((rows, d), lambda i: (i, 0)),
            pl.BlockSpec((d, n), lambda i: (0, 0)),
            pl.BlockSpec((rows, 1), lambda i: (i, 0)),
            pl.BlockSpec((1, n), lambda i: (0, 0)),
        ],
        out_specs=pl.BlockSpec((1, 1), lambda i: (0, 0)),
        out_shape=jax.ShapeDtypeStruct((1, 1), jnp.float32),
    )(x, xt, lab_col, lab_row)
    return out[0, 0]


# speedup vs baseline: 19.9861x; 1.1461x over previous
import functools

import jax
import jax.numpy as jnp
from jax import lax
from jax.experimental import pallas as pl

MARGIN = 0.2
NEG = float("-inf")


def _loss_body(x_ref, xt_ref, labc_ref, labr_ref, start_ref, out_ref):
    i = pl.program_id(0)

    @pl.when(i == 0)
    def _():
        out_ref[...] = jnp.zeros_like(out_ref)

    x = x_ref[...]
    xt = xt_ref[...]
    R = x.shape[0]
    N = xt.shape[1]
    Lf = jnp.float32(N)

    g = jnp.dot(x, xt, preferred_element_type=jnp.float32)
    sqc = jnp.sum(x * x, axis=1, keepdims=True)
    sqr = jnp.sum(xt * xt, axis=0, keepdims=True)
    d2 = jnp.maximum(sqc + sqr - 2.0 * g, 0.0)
    dist = jnp.sqrt(jnp.maximum(d2, 1e-12))
    match = labc_ref[...] == labr_ref[...]
    mf = match.astype(jnp.float32)
    v = -dist + MARGIN * (1.0 - mf)

    kf = jnp.sum(mf, axis=1, keepdims=True)
    ki = kf.astype(jnp.int32)
    kmax = jnp.max(ki)
    iota = lax.broadcasted_iota(jnp.int32, (R, N), 1).astype(jnp.float32)
    startc = start_ref[...]

    def step(t, carry):
        acc, v1 = carry
        r = t + 1
        tf = r.astype(jnp.float32) - 1.0
        active = r <= ki
        m1 = jnp.max(v1, axis=1, keepdims=True)
        w1 = 0.5 + 0.5 * (kf - tf) / kf
        acc = acc + jnp.sum(jnp.where(active, w1 * m1, 0.0))
        i1 = jnp.min(jnp.where(v1 == m1, iota, Lf), axis=1, keepdims=True)
        v1 = jnp.where(iota == i1, NEG, v1)
        pos = startc + tf
        m2 = jnp.sum(jnp.where(iota == pos, v, 0.0), axis=1, keepdims=True)
        cnt_gt = jnp.sum((v > m2).astype(jnp.float32), axis=1, keepdims=True)
        cnt_eq = jnp.sum(((v == m2) & (iota < pos)).astype(jnp.float32),
                         axis=1, keepdims=True)
        rank2 = cnt_gt + cnt_eq + 1.0
        wfp = 0.5 + 0.5 * (kf - rank2 + 1.0) / kf
        wfn = 0.5 + 0.5 * (rank2 - kf) / (Lf - kf)
        w2 = jnp.where(rank2 <= kf, wfp, wfn)
        acc = acc - jnp.sum(jnp.where(active, w2 * m2, 0.0))
        return acc, v1

    acc, _ = lax.fori_loop(0, kmax, step, (jnp.float32(0.0), v))
    out_ref[...] = out_ref[...] + acc


@jax.jit
def kernel(batch_reprs, batch_labels):
    x = batch_reprs.astype(jnp.float32)
    n, d = x.shape
    labels = batch_labels.reshape(-1)
    counts = jnp.zeros((512,), jnp.int32).at[labels].add(1)
    k_row = counts[labels]
    key = k_row * 4096 + labels
    order = jnp.argsort(key)
    skey = key[order]
    start = jnp.searchsorted(skey, skey, side="left").astype(jnp.float32)
    x = x[order]
    labf = labels[order].astype(jnp.float32)
    lab_col = labf.reshape(n, 1)
    lab_row = labf.reshape(1, n)
    xt = x.T

    rows = 128 if n % 128 == 0 else n
    nblk = n // rows

    out = pl.pallas_call(
        _loss_body,
        grid=(nblk,),
        in_specs=[
            pl.BlockSpec((rows, d), lambda i: (i, 0)),
            pl.BlockSpec((d, n), lambda i: (0, 0)),
            pl.BlockSpec((rows, 1), lambda i: (i, 0)),
            pl.BlockSpec((1, n), lambda i: (0, 0)),
            pl.BlockSpec((rows, 1), lambda i: (i, 0)),
        ],
        out_specs=pl.BlockSpec((1, 1), lambda i: (0, 0)),
        out_shape=jax.ShapeDtypeStruct((1, 1), jnp.float32),
    )(x, xt, lab_col, lab_row, start.reshape(n, 1))
    return out[0, 0]
